# baseline (speedup 1.0000x reference)
import jax
import jax.numpy as jnp
import numpy as np
from jax.experimental import pallas as pl
from jax.experimental.pallas import tpu as pltpu
from jax.sharding import Mesh, PartitionSpec as P


def _mm_block(a_ref, w_ref, o_ref):
    o_ref[...] = jnp.dot(a_ref[...], w_ref[...], preferred_element_type=jnp.float32)


def _matmul(ef, w, bm):
    m, k = ef.shape
    _, n = w.shape
    return pl.pallas_call(
        _mm_block,
        grid=(m // bm,),
        compiler_params=pltpu.CompilerParams(
            dimension_semantics=("parallel",),
        ),
        in_specs=[
            pl.BlockSpec((bm, k), lambda i: (i, 0)),
            pl.BlockSpec((k, n), lambda i: (0, 0)),
        ],
        out_specs=pl.BlockSpec((bm, n), lambda i: (i, 0)),
        out_shape=jax.ShapeDtypeStruct((m, n), jnp.float32),
    )(ef, w)


def kernel(explicit_feedback, dense_weight_slice, item_ids):
    m, _ = explicit_feedback.shape
    bm = 512
    devs = jax.devices()
    nd = len(devs)
    if nd > 1 and m % (nd * bm) == 0:
        mesh = Mesh(np.array(devs), ("x",))
        return jax.shard_map(
            lambda a, b: _matmul(a, b, bm),
            mesh=mesh,
            in_specs=(P("x", None), P(None, None)),
            out_specs=P("x", None),
            check_vma=False,
        )(explicit_feedback, dense_weight_slice)
    return _matmul(explicit_feedback, dense_weight_slice, bm)


# speedup vs baseline: 13.4634x; 13.4634x over previous
import jax
import jax.numpy as jnp
from jax.experimental import pallas as pl
from jax.experimental.pallas import tpu as pltpu


def _mm_block(a_ref, w_ref, o_ref):
    o_ref[...] = jax.lax.dot_general(
        a_ref[...], w_ref[...],
        dimension_numbers=(((1,), (0,)), ((), ())),
        preferred_element_type=jnp.float32,
    )


def kernel(explicit_feedback, dense_weight_slice, item_ids):
    m, k = explicit_feedback.shape
    _, n = dense_weight_slice.shape
    w16 = dense_weight_slice.astype(jnp.bfloat16)
    bm = 512
    out = pl.pallas_call(
        _mm_block,
        grid=(m // bm,),
        compiler_params=pltpu.CompilerParams(
            dimension_semantics=("parallel",),
        ),
        in_specs=[
            pl.BlockSpec((bm, k), lambda i: (i, 0)),
            pl.BlockSpec((k, n), lambda i: (0, 0)),
        ],
        out_specs=pl.BlockSpec((bm, n), lambda i: (i, 0)),
        out_shape=jax.ShapeDtypeStruct((m, n), jnp.float32),
    )(explicit_feedback, w16)
    return out


# speedup vs baseline: 14.9497x; 1.1104x over previous
import jax
import jax.numpy as jnp
from jax.experimental import pallas as pl
from jax.experimental.pallas import tpu as pltpu


def _mm_block(a_ref, w_ref, o_ref):
    o_ref[...] = jnp.dot(a_ref[...], w_ref[...], preferred_element_type=jnp.float32)


def kernel(explicit_feedback, dense_weight_slice, item_ids):
    m, k = explicit_feedback.shape
    _, n = dense_weight_slice.shape
    bm = 512
    out = pl.pallas_call(
        _mm_block,
        grid=(m // bm,),
        compiler_params=pltpu.CompilerParams(
            dimension_semantics=("parallel",),
        ),
        in_specs=[
            pl.BlockSpec((bm, k), lambda i: (i, 0)),
            pl.BlockSpec((k, n), lambda i: (0, 0)),
        ],
        out_specs=pl.BlockSpec((bm, n), lambda i: (i, 0)),
        out_shape=jax.ShapeDtypeStruct((m, n), jnp.float32),
    )(explicit_feedback, dense_weight_slice)
    return out
